# baseline (speedup 1.0000x reference)
import functools

import jax
import jax.numpy as jnp
from jax import lax
from jax.experimental import pallas as pl
from jax.experimental.pallas import tpu as pltpu
from jax.experimental.pallas import tpu_sc as plsc

NC = 2
NS = 16
CH = 120
KW = 8
ZPAD = 8


def _sc_aggregate(n_nodes, d, n_win):
    chunk = (n_nodes // NS) // 8 * 8
    last = n_nodes - (NS - 1) * chunk
    d2 = d // 2
    n_grp = n_win // KW
    mesh = plsc.VectorSubcoreMesh(
        core_axis_name="c", subcore_axis_name="s", num_cores=NC,
        num_subcores=NS)

    @functools.partial(
        pl.kernel,
        out_type=jax.ShapeDtypeStruct((NC, n_nodes, d), jnp.float32),
        mesh=mesh,
        scratch_types=[
            pltpu.VMEM((KW, CH), jnp.int32),
            pltpu.VMEM((KW, CH), jnp.int32),
            pltpu.VMEM((KW, CH), jnp.int32),
            pltpu.VMEM((KW, CH), jnp.int32),
            pltpu.VMEM((CH, d // 2), jnp.int32),
            pltpu.VMEM((CH, d // 2), jnp.int32),
            pltpu.VMEM((CH, d), jnp.float32),
            pltpu.VMEM((CH, d), jnp.float32),
            pltpu.SemaphoreType.DMA,
            pltpu.SemaphoreType.DMA,
            pltpu.SemaphoreType.DMA,
            pltpu.SemaphoreType.DMA,
            pltpu.VMEM_SHARED((n_nodes, d), jnp.float32),
        ],
        compiler_params=pltpu.CompilerParams(use_tc_tiling_on_sc=False,
                                             needs_layout_passes=False),
    )
    def sc_kernel(nodes_hbm, a_hbm, b_hbm, out_hbm, a_va, b_va, a_vb, b_vb,
                  bf0_v, bf1_v, f0_v, f1_v, sem_g0, sem_g1, sem_s, sem_i,
                  agg_sh):
        cid = lax.axis_index("c")
        sid = lax.axis_index("s")

        def zero_row(i, carry):
            z = jnp.zeros((16,), jnp.float32)
            for jj in range(d // 16):
                f0_v[i, pl.ds(jj * 16, 16)] = z
            return carry
        lax.fori_loop(0, CH, zero_row, 0)

        base = pl.multiple_of(sid * chunk, 8)

        def zero_span(start, count):
            full, rem = divmod(count, CH)
            for t in range(full):
                pltpu.sync_copy(f0_v, agg_sh.at[pl.ds(start + t * CH, CH)])
            if rem:
                pltpu.sync_copy(f0_v.at[pl.ds(0, rem)],
                                agg_sh.at[pl.ds(start + full * CH, rem)])

        @pl.when(sid < NS - 1)
        def _():
            zero_span(base, chunk)

        @pl.when(sid == NS - 1)
        def _():
            zero_span(base, last)

        plsc.subcore_barrier()

        def stage_idx(g, a_v, b_v):
            goff = pl.multiple_of(g * KW, KW)
            pltpu.async_copy(a_hbm.at[cid, sid, pl.ds(goff, KW)], a_v, sem_i)
            pltpu.async_copy(b_hbm.at[cid, sid, pl.ds(goff, KW)], b_v, sem_i)

        def wait_idx(a_v, b_v):
            pltpu.make_async_copy(a_hbm.at[cid, sid, pl.ds(0, KW)], a_v,
                                  sem_i).wait()
            pltpu.make_async_copy(b_hbm.at[cid, sid, pl.ds(0, KW)], b_v,
                                  sem_i).wait()

        def unpack_win(bf_v, f_v):
            def row(r, carry):
                for c in range(d2 // 16):
                    v = bf_v[r, pl.ds(c * 16, 16)]
                    lo = plsc.bitcast(lax.shift_left(v, 16), jnp.float32)
                    hi = plsc.bitcast(
                        jnp.bitwise_and(v, jnp.int32(-65536)), jnp.float32)
                    f_v[r, pl.ds(c * 16, 16)] = lo
                    f_v[r, pl.ds(d2 + c * 16, 16)] = hi
                return carry
            lax.fori_loop(0, CH, row, 0, unroll=4)

        def drain_scatter():
            pltpu.make_async_copy(f0_v, agg_sh.at[pl.ds(0, CH)],
                                  sem_s).wait()

        def process_group(g, a_v, b_v, a_nxt, b_nxt, has_next):
            def half(j, bf_cur, f_cur, bf_oth, sem_cur, sem_oth,
                     static_last=False):
                if static_last:
                    @pl.when(has_next)
                    def _():
                        wait_idx(a_nxt, b_nxt)
                        pltpu.async_copy(nodes_hbm.at[b_nxt.at[0]], bf_oth,
                                         sem_oth)
                else:
                    @pl.when(j < KW - 1)
                    def _():
                        pltpu.async_copy(nodes_hbm.at[b_v.at[j + 1]],
                                         bf_oth, sem_oth)
                pltpu.make_async_copy(nodes_hbm.at[b_v.at[j]], bf_cur,
                                      sem_cur).wait()
                unpack_win(bf_cur, f_cur)

                @pl.when(jnp.logical_or(g > 0, j > 1))
                def _():
                    drain_scatter()
                pltpu.async_copy(f_cur, agg_sh.at[a_v.at[j]], sem_s,
                                 add=True)

            def pair(p, carry2):
                j0 = 2 * p
                half(j0, bf0_v, f0_v, bf1_v, sem_g0, sem_g1)
                half(j0 + 1, bf1_v, f1_v, bf0_v, sem_g1, sem_g0)
                return carry2
            lax.fori_loop(0, KW // 2 - 1, pair, 0)
            half(KW - 2, bf0_v, f0_v, bf1_v, sem_g0, sem_g1)
            half(KW - 1, bf1_v, f1_v, bf0_v, sem_g1, sem_g0,
                 static_last=True)

        stage_idx(0, a_va, b_va)
        wait_idx(a_va, b_va)
        pltpu.async_copy(nodes_hbm.at[b_va.at[0]], bf0_v, sem_g0)

        def outer(g, carry):
            @pl.when(g % 2 == 0)
            def _():
                @pl.when(g + 1 < n_grp)
                def _():
                    stage_idx(g + 1, a_vb, b_vb)
                process_group(g, a_va, b_va, a_vb, b_vb, g + 1 < n_grp)

            @pl.when(g % 2 == 1)
            def _():
                @pl.when(g + 1 < n_grp)
                def _():
                    stage_idx(g + 1, a_va, b_va)
                process_group(g, a_vb, b_vb, a_va, b_va, g + 1 < n_grp)
            return carry
        lax.fori_loop(0, n_grp, outer, 0)

        drain_scatter()
        drain_scatter()
        plsc.subcore_barrier()

        @pl.when(sid < NS - 1)
        def _():
            pltpu.sync_copy(agg_sh.at[pl.ds(base, chunk)],
                            out_hbm.at[cid, pl.ds(base, chunk)])

        @pl.when(sid == NS - 1)
        def _():
            pltpu.sync_copy(agg_sh.at[pl.ds(base, last)],
                            out_hbm.at[cid, pl.ds(base, last)])

    return sc_kernel


def _dense_body(p_ref, x_ref, dn_ref, nn_ref, w_ref, o_ref):
    agg = p_ref[0] + p_ref[1]
    h = agg * nn_ref[...] + x_ref[...] * dn_ref[...]
    o_ref[...] = jnp.maximum(
        jnp.dot(h, w_ref[...].T, preferred_element_type=jnp.float32), 0.0)


def kernel(nodes, edge_index, degrees, normalizers, W):
    n, d = nodes.shape
    e = edge_index.shape[0]

    src = edge_index[:, 0]
    dst = edge_index[:, 1]
    e2 = 2 * e
    n_win = -(-e2 // (NC * NS * CH))
    n_win = -(-n_win // KW) * KW
    pad = NC * NS * n_win * CH - e2
    pad_ar = jnp.arange(pad, dtype=jnp.int32)
    a_idx = jnp.concatenate([src, dst, pad_ar % n])
    b_idx = jnp.concatenate([dst, src, n + (pad_ar % ZPAD)])
    a_idx = a_idx.reshape(NC, NS, n_win, CH)
    b_idx = b_idx.reshape(NC, NS, n_win, CH)

    nodes_bf = nodes.astype(jnp.bfloat16).reshape(n, d // 2, 2)
    nodes_i32 = jax.lax.bitcast_convert_type(nodes_bf, jnp.int32)
    nodes_i32 = jnp.concatenate(
        [nodes_i32, jnp.zeros((ZPAD, d // 2), jnp.int32)])

    partials = _sc_aggregate(n, d, n_win)(nodes_i32, a_idx, b_idx)

    perm = jnp.concatenate([jnp.arange(0, d, 2), jnp.arange(1, d, 2)])
    nodes_p = nodes[:, perm]
    w_p = W[:, perm]

    inv_deg = (1.0 / degrees).reshape(n, 1)
    inv_norm = (1.0 / normalizers).reshape(n, 1)

    out = pl.pallas_call(
        _dense_body,
        out_shape=jax.ShapeDtypeStruct((n, d), jnp.float32),
    )(partials, nodes_p, inv_deg, inv_norm, w_p)
    return out


# speedup vs baseline: 1.9250x; 1.9250x over previous
import functools

import jax
import jax.numpy as jnp
from jax import lax
from jax.experimental import pallas as pl
from jax.experimental.pallas import tpu as pltpu
from jax.experimental.pallas import tpu_sc as plsc

NC = 2
NS = 16
CH = 120
KW = 6
ZPAD = 8


def _sc_aggregate(n_nodes, d, n_win):
    chunk = (n_nodes // NS) // 8 * 8
    last = n_nodes - (NS - 1) * chunk
    n_grp = n_win // KW
    mesh = plsc.VectorSubcoreMesh(
        core_axis_name="c", subcore_axis_name="s", num_cores=NC,
        num_subcores=NS)

    @functools.partial(
        pl.kernel,
        out_type=jax.ShapeDtypeStruct((NC, n_nodes, d), jnp.float32),
        mesh=mesh,
        scratch_types=[
            pltpu.VMEM((KW, CH), jnp.int32),
            pltpu.VMEM((KW, CH), jnp.int32),
            pltpu.VMEM((KW, CH), jnp.int32),
            pltpu.VMEM((KW, CH), jnp.int32),
            pltpu.VMEM((CH, d), jnp.float32),
            pltpu.VMEM((CH, d), jnp.float32),
            pltpu.VMEM((CH, d), jnp.float32),
            pltpu.SemaphoreType.DMA,
            pltpu.SemaphoreType.DMA,
            pltpu.SemaphoreType.DMA,
            pltpu.SemaphoreType.DMA,
            pltpu.VMEM_SHARED((n_nodes, d), jnp.float32),
        ],
        compiler_params=pltpu.CompilerParams(use_tc_tiling_on_sc=False),
    )
    def sc_kernel(nodes_hbm, a_hbm, b_hbm, out_hbm, a_va, b_va, a_vb, b_vb,
                  r0_v, r1_v, r2_v, sem0, sem1, sem2, sem_i, agg_sh):
        cid = lax.axis_index("c")
        sid = lax.axis_index("s")
        rings = [(r0_v, sem0), (r1_v, sem1), (r2_v, sem2)]

        def zero_row(i, carry):
            z = jnp.zeros((16,), jnp.float32)
            for jj in range(d // 16):
                r0_v[i, pl.ds(jj * 16, 16)] = z
            return carry
        lax.fori_loop(0, CH, zero_row, 0)

        base = pl.multiple_of(sid * chunk, 8)

        def zero_span(start, count):
            full, rem = divmod(count, CH)
            for t in range(full):
                pltpu.sync_copy(r0_v, agg_sh.at[pl.ds(start + t * CH, CH)])
            if rem:
                pltpu.sync_copy(r0_v.at[pl.ds(0, rem)],
                                agg_sh.at[pl.ds(start + full * CH, rem)])

        @pl.when(sid < NS - 1)
        def _():
            zero_span(base, chunk)

        @pl.when(sid == NS - 1)
        def _():
            zero_span(base, last)

        plsc.subcore_barrier()

        def stage_idx(g, a_v, b_v):
            goff = pl.multiple_of(g * KW, KW)
            pltpu.async_copy(a_hbm.at[cid, sid, pl.ds(goff, KW)], a_v, sem_i)
            pltpu.async_copy(b_hbm.at[cid, sid, pl.ds(goff, KW)], b_v, sem_i)

        def wait_idx(a_v, b_v):
            pltpu.make_async_copy(a_hbm.at[cid, sid, pl.ds(0, KW)], a_v,
                                  sem_i).wait()
            pltpu.make_async_copy(b_hbm.at[cid, sid, pl.ds(0, KW)], b_v,
                                  sem_i).wait()

        def gather(b_v, j, buf, sem):
            pltpu.async_copy(nodes_hbm.at[b_v.at[j]], buf, sem)

        def process_group(a_v, b_v, a_nxt, b_nxt, has_next):
            for j in range(KW):
                nxt = j + 2
                nbuf, nsem = rings[nxt % 3]
                if nxt < KW:
                    gather(b_v, nxt, nbuf, nsem)
                elif nxt == KW:
                    @pl.when(has_next)
                    def _():
                        wait_idx(a_nxt, b_nxt)
                        gather(b_nxt, 0, nbuf, nsem)
                else:
                    @pl.when(has_next)
                    def _():
                        gather(b_nxt, 1, nbuf, nsem)
                buf, sem = rings[j % 3]
                pltpu.make_async_copy(nodes_hbm.at[pl.ds(0, CH)], buf,
                                      sem).wait()
                pltpu.sync_copy(buf, agg_sh.at[a_v.at[j]], add=True)

        stage_idx(0, a_va, b_va)
        wait_idx(a_va, b_va)
        gather(b_va, 0, r0_v, sem0)
        gather(b_va, 1, r1_v, sem1)

        def outer(g, carry):
            @pl.when(g % 2 == 0)
            def _():
                @pl.when(g + 1 < n_grp)
                def _():
                    stage_idx(g + 1, a_vb, b_vb)
                process_group(a_va, b_va, a_vb, b_vb, g + 1 < n_grp)

            @pl.when(g % 2 == 1)
            def _():
                @pl.when(g + 1 < n_grp)
                def _():
                    stage_idx(g + 1, a_va, b_va)
                process_group(a_vb, b_vb, a_va, b_va, g + 1 < n_grp)
            return carry
        lax.fori_loop(0, n_grp, outer, 0)

        plsc.subcore_barrier()

        @pl.when(sid < NS - 1)
        def _():
            pltpu.sync_copy(agg_sh.at[pl.ds(base, chunk)],
                            out_hbm.at[cid, pl.ds(base, chunk)])

        @pl.when(sid == NS - 1)
        def _():
            pltpu.sync_copy(agg_sh.at[pl.ds(base, last)],
                            out_hbm.at[cid, pl.ds(base, last)])

    return sc_kernel


def _dense_body(p_ref, x_ref, dn_ref, nn_ref, w_ref, o_ref):
    agg = p_ref[0] + p_ref[1]
    h = agg * nn_ref[...] + x_ref[...] * dn_ref[...]
    o_ref[...] = jnp.maximum(
        jnp.dot(h, w_ref[...].T, preferred_element_type=jnp.float32), 0.0)


def kernel(nodes, edge_index, degrees, normalizers, W):
    n, d = nodes.shape
    e = edge_index.shape[0]

    src = edge_index[:, 0]
    dst = edge_index[:, 1]
    e2 = 2 * e
    n_win = -(-e2 // (NC * NS * CH))
    n_win = -(-n_win // KW) * KW
    pad = NC * NS * n_win * CH - e2
    pad_ar = jnp.arange(pad, dtype=jnp.int32)
    a_idx = jnp.concatenate([src, dst, pad_ar % n])
    b_idx = jnp.concatenate([dst, src, n + (pad_ar % ZPAD)])
    a_idx = a_idx.reshape(NC, NS, n_win, CH)
    b_idx = b_idx.reshape(NC, NS, n_win, CH)

    nodes_ext = jnp.concatenate([nodes, jnp.zeros((ZPAD, d), nodes.dtype)])
    partials = _sc_aggregate(n, d, n_win)(nodes_ext, a_idx, b_idx)

    inv_deg = (1.0 / degrees).reshape(n, 1)
    inv_norm = (1.0 / normalizers).reshape(n, 1)

    out = pl.pallas_call(
        _dense_body,
        out_shape=jax.ShapeDtypeStruct((n, d), jnp.float32),
    )(partials, nodes, inv_deg, inv_norm, W)
    return out


# speedup vs baseline: 2.1767x; 1.1308x over previous
import functools

import jax
import jax.numpy as jnp
from jax import lax
from jax.experimental import pallas as pl
from jax.experimental.pallas import tpu as pltpu
from jax.experimental.pallas import tpu_sc as plsc

NC = 2
NS = 16
CH = 120
KW = 6
TRASH = 8


def _sc_aggregate(n_nodes, d, n_win):
    chunk = (n_nodes // NS) // 8 * 8
    last = n_nodes - (NS - 1) * chunk
    n_grp = n_win // KW
    mesh = plsc.VectorSubcoreMesh(
        core_axis_name="c", subcore_axis_name="s", num_cores=NC,
        num_subcores=NS)

    @functools.partial(
        pl.kernel,
        out_type=jax.ShapeDtypeStruct((NC, n_nodes, d), jnp.float32),
        mesh=mesh,
        scratch_types=[
            pltpu.VMEM((KW, CH), jnp.int32),
            pltpu.VMEM((KW, CH), jnp.int32),
            pltpu.VMEM((KW, CH), jnp.int32),
            pltpu.VMEM((KW, CH), jnp.int32),
            pltpu.VMEM((CH, d), jnp.float32),
            pltpu.VMEM((CH, d), jnp.float32),
            pltpu.VMEM((CH, d), jnp.float32),
            pltpu.SemaphoreType.DMA,
            pltpu.SemaphoreType.DMA,
            pltpu.SemaphoreType.DMA,
            pltpu.SemaphoreType.DMA,
            pltpu.VMEM_SHARED((n_nodes + TRASH, d), jnp.float32),
        ],
        compiler_params=pltpu.CompilerParams(use_tc_tiling_on_sc=False),
    )
    def sc_kernel(nodes_hbm, a_hbm, b_hbm, out_hbm, a_va, b_va, a_vb, b_vb,
                  r0_v, r1_v, r2_v, sem0, sem1, sem2, sem_i, agg_sh):
        cid = lax.axis_index("c")
        sid = lax.axis_index("s")
        rings = [(r0_v, sem0), (r1_v, sem1), (r2_v, sem2)]

        def zero_row(i, carry):
            z = jnp.zeros((16,), jnp.float32)
            for jj in range(d // 16):
                r0_v[i, pl.ds(jj * 16, 16)] = z
            return carry
        lax.fori_loop(0, CH, zero_row, 0)

        base = pl.multiple_of(sid * chunk, 8)

        def zero_span(start, count):
            full, rem = divmod(count, CH)
            for t in range(full):
                pltpu.sync_copy(r0_v, agg_sh.at[pl.ds(start + t * CH, CH)])
            if rem:
                pltpu.sync_copy(r0_v.at[pl.ds(0, rem)],
                                agg_sh.at[pl.ds(start + full * CH, rem)])

        @pl.when(sid < NS - 1)
        def _():
            zero_span(base, chunk)

        @pl.when(sid == NS - 1)
        def _():
            zero_span(base, last + TRASH)

        plsc.subcore_barrier()

        def stage_idx(g, a_v, b_v):
            goff = pl.multiple_of(g * KW, KW)
            pltpu.async_copy(a_hbm.at[cid, sid, pl.ds(goff, KW)], a_v, sem_i)
            pltpu.async_copy(b_hbm.at[cid, sid, pl.ds(goff, KW)], b_v, sem_i)

        def wait_idx(a_v, b_v):
            pltpu.make_async_copy(a_hbm.at[cid, sid, pl.ds(0, KW)], a_v,
                                  sem_i).wait()
            pltpu.make_async_copy(b_hbm.at[cid, sid, pl.ds(0, KW)], b_v,
                                  sem_i).wait()

        def gather(b_v, j, buf, sem):
            pltpu.async_copy(nodes_hbm.at[b_v.at[j]], buf, sem)

        def process_group(a_v, b_v, a_nxt, b_nxt, has_next):
            for j in range(KW):
                nxt = j + 2
                nbuf, nsem = rings[nxt % 3]
                if nxt < KW:
                    gather(b_v, nxt, nbuf, nsem)
                elif nxt == KW:
                    @pl.when(has_next)
                    def _():
                        wait_idx(a_nxt, b_nxt)
                        gather(b_nxt, 0, nbuf, nsem)
                else:
                    @pl.when(has_next)
                    def _():
                        gather(b_nxt, 1, nbuf, nsem)
                buf, sem = rings[j % 3]
                pltpu.make_async_copy(nodes_hbm.at[pl.ds(0, CH)], buf,
                                      sem).wait()
                pltpu.sync_copy(buf, agg_sh.at[a_v.at[j]], add=True)

        stage_idx(0, a_va, b_va)
        wait_idx(a_va, b_va)
        gather(b_va, 0, r0_v, sem0)
        gather(b_va, 1, r1_v, sem1)

        def outer(g, carry):
            @pl.when(g % 2 == 0)
            def _():
                @pl.when(g + 1 < n_grp)
                def _():
                    stage_idx(g + 1, a_vb, b_vb)
                process_group(a_va, b_va, a_vb, b_vb, g + 1 < n_grp)

            @pl.when(g % 2 == 1)
            def _():
                @pl.when(g + 1 < n_grp)
                def _():
                    stage_idx(g + 1, a_va, b_va)
                process_group(a_vb, b_vb, a_va, b_va, g + 1 < n_grp)
            return carry
        lax.fori_loop(0, n_grp, outer, 0)

        plsc.subcore_barrier()

        @pl.when(sid < NS - 1)
        def _():
            pltpu.sync_copy(agg_sh.at[pl.ds(base, chunk)],
                            out_hbm.at[cid, pl.ds(base, chunk)])

        @pl.when(sid == NS - 1)
        def _():
            pltpu.sync_copy(agg_sh.at[pl.ds(base, last)],
                            out_hbm.at[cid, pl.ds(base, last)])

    return sc_kernel


def _dense_body(p_ref, x_ref, dn_ref, nn_ref, w_ref, o_ref):
    agg = p_ref[0] + p_ref[1]
    h = agg * nn_ref[...] + x_ref[...] * dn_ref[...]
    o_ref[...] = jnp.maximum(
        jnp.dot(h, w_ref[...].T, preferred_element_type=jnp.float32), 0.0)


def kernel(nodes, edge_index, degrees, normalizers, W):
    n, d = nodes.shape
    e = edge_index.shape[0]

    src = edge_index[:, 0]
    dst = edge_index[:, 1]
    e2 = 2 * e
    n_win = -(-e2 // (NC * NS * CH))
    n_win = -(-n_win // KW) * KW
    pad = NC * NS * n_win * CH - e2
    pad_ar = jnp.arange(pad, dtype=jnp.int32)
    a_idx = jnp.concatenate([src, dst, n + (pad_ar % TRASH)])
    b_idx = jnp.concatenate([dst, src, pad_ar % n])
    a_idx = a_idx.reshape(NC, NS, n_win, CH)
    b_idx = b_idx.reshape(NC, NS, n_win, CH)

    partials = _sc_aggregate(n, d, n_win)(nodes, a_idx, b_idx)

    inv_deg = (1.0 / degrees).reshape(n, 1)
    inv_norm = (1.0 / normalizers).reshape(n, 1)

    out = pl.pallas_call(
        _dense_body,
        out_shape=jax.ShapeDtypeStruct((n, d), jnp.float32),
    )(partials, nodes, inv_deg, inv_norm, W)
    return out
